# baseline (speedup 1.0000x reference)
import dataclasses
import functools

import jax
import jax.numpy as jnp
from jax import lax
from jax.experimental import pallas as pl
from jax.experimental.pallas import tpu as pltpu
from jax.experimental.pallas import tpu_sc as plsc

N = 10000
D = 128
E = 320000
NC, NS = 2, 16
NW = NC * NS
CHUNK = 64
NBUF = 4
TPW = 160
E_TILE = CHUNK * TPW
E_PAD = E_TILE * NW
NCH = E_PAD // CHUNK
ROWS_Z = 632
N_ACC = NS * ROWS_Z
IG = 8
CR = 80


_MESH = plsc.VectorSubcoreMesh(core_axis_name="c", subcore_axis_name="s")

_f32 = jnp.float32

_SC_PARAMS = dataclasses.replace(
    pltpu.CompilerParams(), needs_layout_passes=False,
    use_tc_tiling_on_sc=False)

DH = D // 2
RPT = N // NS


@functools.partial(
    pl.kernel,
    out_type=(
        jax.ShapeDtypeStruct((NC, N_ACC, D), _f32),
        jax.ShapeDtypeStruct((NC, CR, 128), _f32),
        jax.ShapeDtypeStruct((NC, N_ACC, D), _f32),
        jax.ShapeDtypeStruct((NC, CR, 128), _f32),
        jax.ShapeDtypeStruct((NC, N_ACC, D), _f32),
        jax.ShapeDtypeStruct((NC, CR, 128), _f32),
    ),
    mesh=_MESH,
    compiler_params=_SC_PARAMS,
    scratch_types=[
        pltpu.VMEM((IG, CHUNK), jnp.int32),
        pltpu.VMEM((IG, CHUNK), jnp.int32),
        pltpu.VMEM((NBUF, CHUNK, DH // 2), jnp.int32),
        pltpu.VMEM((NBUF, CHUNK, DH), _f32),
        pltpu.VMEM((CR, 128), _f32),
        pltpu.VMEM((1, CR), jnp.int32),
        pltpu.VMEM_SHARED((N, DH // 2), jnp.int32),
        pltpu.VMEM_SHARED((N_ACC, DH), _f32),
        pltpu.VMEM_SHARED((CR, 128), _f32),
        pltpu.SemaphoreType.DMA,
        pltpu.SemaphoreType.DMA,
        pltpu.SemaphoreType.DMA,
        pltpu.SemaphoreType.DMA,
        pltpu.SemaphoreType.DMA,
        pltpu.SemaphoreType.DMA,
        pltpu.SemaphoreType.DMA,
        pltpu.SemaphoreType.DMA,
    ],
)
def _sc_aggregate(fu, fi, s0, d0, s1, d1, s2, d2, za, irow,
                  sum0, cnt0, sum1, cnt1, sum2, cnt2,
                  src_v, dst_v, rows_p, rows_f, lcnt, irow_v,
                  tbl_sh, acc_sh, cnt_sh,
                  sem_a, sem_b, sem_c, sem_d, sem_e, sem_f, sem_g, sem_h):
    c = lax.axis_index("c")
    s = lax.axis_index("s")
    base = (c * NS + s) * TPW
    ones16 = jnp.full((16,), 1.0, _f32)
    sems = (sem_a, sem_b, sem_c, sem_d)
    ssems = (sem_e, sem_f, sem_g, sem_h)
    pltpu.sync_copy(irow, irow_v)
    for tbl, src_h, dst_h, sum_o, cnt_o in (
        (fu, s0, d0, sum0, cnt0),
        (fu, s1, d1, sum1, cnt1),
        (fi, s2, d2, sum2, cnt2),
    ):
        for h in (0, 1):
            HW = DH // 2
            pltpu.sync_copy(tbl.at[pl.ds(s * RPT, RPT), pl.ds(h * HW, HW)],
                            tbl_sh.at[pl.ds(s * RPT, RPT)])
            pltpu.sync_copy(za.at[pl.ds(0, ROWS_Z), pl.ds(0, DH)],
                            acc_sh.at[pl.ds(s * ROWS_Z, ROWS_Z)])
            if h == 0:
                pltpu.sync_copy(za.at[pl.ds(0, CR)], lcnt)

                @pl.when(s < CR // 8)
                def _():
                    pltpu.sync_copy(za.at[pl.ds(0, 8)],
                                    cnt_sh.at[pl.ds(s * 8, 8)])

            plsc.subcore_barrier()

            @pl.loop(0, TPW // IG)
            def _(g):
                off = base + g * IG
                pltpu.sync_copy(src_h.at[pl.ds(off, IG)], src_v)
                pltpu.sync_copy(dst_h.at[pl.ds(off, IG)], dst_v)
                desc = [None] * NBUF
                sdesc = [None] * NBUF
                for p in range(NBUF - 1):
                    desc[p] = pltpu.async_copy(tbl_sh.at[src_v.at[p]],
                                               rows_p.at[p], sems[p])
                for j in range(IG):
                    cur = j % NBUF
                    if j + NBUF - 1 < IG:
                        nxt = (j + NBUF - 1) % NBUF
                        if sdesc[nxt] is not None:
                            sdesc[nxt].wait()
                            sdesc[nxt] = None
                        desc[nxt] = pltpu.async_copy(
                            tbl_sh.at[src_v.at[j + NBUF - 1]], rows_p.at[nxt],
                            sems[nxt])
                    if h == 0:
                        for k in range(CHUNK // 16):
                            v = dst_v[j, pl.ds(k * 16, 16)]
                            row = lax.shift_right_logical(v, 7)
                            col = jnp.bitwise_and(v, 127)
                            plsc.addupdate_scatter(lcnt, [row, col], ones16)
                    desc[cur].wait()

                    @pl.loop(0, CHUNK // 8)
                    def _(u):
                        for rr in range(8):
                            for q in range(2):
                                w = rows_p[cur, u * 8 + rr,
                                           pl.ds(q * 16, 16)]
                                bf = plsc.bitcast(w, jnp.bfloat16)
                                aa, bb = plsc.unpack(
                                    bf, format=plsc.PackFormat.INTERLEAVED,
                                    preferred_element_type=_f32)
                                rows_f[cur, u * 8 + rr,
                                       pl.ds(q * 32, 16)] = aa
                                rows_f[cur, u * 8 + rr,
                                       pl.ds(q * 32 + 16, 16)] = bb

                    sdesc[cur] = pltpu.async_copy(
                        rows_f.at[cur], acc_sh.at[dst_v.at[j]], ssems[cur],
                        add=True)
                for bq in range(NBUF):
                    if sdesc[bq] is not None:
                        sdesc[bq].wait()

            if h == 0:
                pltpu.sync_copy(lcnt, cnt_sh.at[irow_v.at[0]], add=True)
            plsc.subcore_barrier()
            pltpu.sync_copy(acc_sh.at[pl.ds(s * ROWS_Z, ROWS_Z)],
                            sum_o.at[c, pl.ds(s * ROWS_Z, ROWS_Z),
                                     pl.ds(h * DH, DH)])
            if h == 0:

                @pl.when(s == 0)
                def _():
                    pltpu.sync_copy(cnt_sh, cnt_o.at[c])

            plsc.subcore_barrier()


BLK = 400


def _combine_body(sf, cf, srb, crb, sr, cr, wf, bf, wrb, brb, wr, br,
                  hu_ref, hi_ref):
    def mean_of(s_ref, c_ref):
        t = s_ref[0] + s_ref[1]
        cnt = c_ref[0] + c_ref[1]
        return t / jnp.maximum(cnt, 1.0), cnt

    dn = (((1,), (1,)), ((), ()))
    mf, cntf = mean_of(sf, cf)
    mrb, cntrb = mean_of(srb, crb)
    mr, cntr = mean_of(sr, cr)
    hu_ref[...] = (
        lax.dot_general(mf, wf[...], dn, preferred_element_type=_f32)
        + jnp.where(cntf > 0, bf[...], 0.0)
        + lax.dot_general(mrb, wrb[...], dn, preferred_element_type=_f32)
        + jnp.where(cntrb > 0, brb[...], 0.0)
    )
    hi_ref[...] = (
        lax.dot_general(mr, wr[...], dn, preferred_element_type=_f32)
        + jnp.where(cntr > 0, br[...], 0.0)
    )


def _combine(sf, cf, srb, crb, sr, cr, wf, bf, wrb, brb, wr, br):
    sum_spec = pl.BlockSpec((NC, BLK, D), lambda i: (0, i, 0))
    cnt_spec = pl.BlockSpec((NC, BLK, 1), lambda i: (0, i, 0))
    w_spec = pl.BlockSpec((D, D), lambda i: (0, 0))
    b_spec = pl.BlockSpec((1, D), lambda i: (0, 0))
    out_spec = pl.BlockSpec((BLK, D), lambda i: (i, 0))
    return pl.pallas_call(
        _combine_body,
        grid=(N // BLK,),
        in_specs=[sum_spec, cnt_spec, sum_spec, cnt_spec, sum_spec, cnt_spec,
                  w_spec, b_spec, w_spec, b_spec, w_spec, b_spec],
        out_specs=(out_spec, out_spec),
        out_shape=(jax.ShapeDtypeStruct((N, D), _f32),
                   jax.ShapeDtypeStruct((N, D), _f32)),
    )(sf, cf, srb, crb, sr, cr, wf, bf, wrb, brb, wr, br)


def kernel(feat_user, feat_item, W_follows, b_follows, W_rates, b_rates,
           W_rated_by, b_rated_by, src_follows, dst_follows, src_rates,
           dst_rates, src_rated_by, dst_rated_by):
    pad = E_PAD - E

    def prep(src, dst):
        s = jnp.concatenate([src, jnp.zeros((pad,), jnp.int32)])
        d = jnp.concatenate([dst, jnp.full((pad,), N, jnp.int32)])
        return s.reshape(NCH, CHUNK), d.reshape(NCH, CHUNK)

    s0, d0 = prep(src_follows, dst_follows)
    s1, d1 = prep(src_rates, dst_rates)
    s2, d2 = prep(src_rated_by, dst_rated_by)
    za = jnp.zeros((ROWS_Z, D), _f32)
    irow = jnp.arange(CR, dtype=jnp.int32).reshape(1, CR)

    def packh(x):
        xb = x.astype(jnp.bfloat16)
        y = xb.reshape(N, 4, 2, 16).transpose(0, 1, 3, 2)
        return jax.lax.bitcast_convert_type(y, jnp.int32).reshape(N, 64)

    sumf, cntf, sumr, cntr, sumrb, cntrb = _sc_aggregate(
        packh(feat_user), packh(feat_item),
        s0, d0, s1, d1, s2, d2, za, irow)

    def cview(cnt):
        return cnt.reshape(NC, CR * 128)[:, :N, None]

    h_user, h_item = _combine(
        sumf, cview(cntf), sumrb, cview(cntrb), sumr, cview(cntr),
        W_follows, b_follows.reshape(1, D),
        W_rated_by, b_rated_by.reshape(1, D),
        W_rates, b_rates.reshape(1, D))
    return (h_user, h_item)


# speedup vs baseline: 1.8631x; 1.8631x over previous
import dataclasses
import functools

import jax
import jax.numpy as jnp
from jax import lax
from jax.experimental import pallas as pl
from jax.experimental.pallas import tpu as pltpu
from jax.experimental.pallas import tpu_sc as plsc

N = 10000
D = 128
E = 320000
NC, NS = 2, 16
NW = NC * NS
CHUNK = 64
NBUF = 4
TPW = 160
E_TILE = CHUNK * TPW
E_PAD = E_TILE * NW
NCH = E_PAD // CHUNK
ROWS_Z = 632
N_ACC = NS * ROWS_Z
IG = 80
CR = 80


_MESH = plsc.VectorSubcoreMesh(core_axis_name="c", subcore_axis_name="s")

_f32 = jnp.float32

_SC_PARAMS = dataclasses.replace(
    pltpu.CompilerParams(), needs_layout_passes=False,
    use_tc_tiling_on_sc=False)

DH = D // 2
RPT = N // NS


@functools.partial(
    pl.kernel,
    out_type=(
        jax.ShapeDtypeStruct((NC, N_ACC, D), _f32),
        jax.ShapeDtypeStruct((NC, CR, 128), _f32),
        jax.ShapeDtypeStruct((NC, N_ACC, D), _f32),
        jax.ShapeDtypeStruct((NC, CR, 128), _f32),
        jax.ShapeDtypeStruct((NC, N_ACC, D), _f32),
        jax.ShapeDtypeStruct((NC, CR, 128), _f32),
    ),
    mesh=_MESH,
    compiler_params=_SC_PARAMS,
    scratch_types=[
        pltpu.VMEM((IG, CHUNK), jnp.int32),
        pltpu.VMEM((IG, CHUNK), jnp.int32),
        pltpu.VMEM((NBUF, CHUNK, DH), _f32),
        pltpu.VMEM((CR, 128), _f32),
        pltpu.VMEM((1, CR), jnp.int32),
        pltpu.VMEM_SHARED((N, DH), _f32),
        pltpu.VMEM_SHARED((N_ACC, DH), _f32),
        pltpu.VMEM_SHARED((CR, 128), _f32),
        pltpu.SemaphoreType.DMA,
        pltpu.SemaphoreType.DMA,
        pltpu.SemaphoreType.DMA,
        pltpu.SemaphoreType.DMA,
        pltpu.SemaphoreType.DMA,
        pltpu.SemaphoreType.DMA,
        pltpu.SemaphoreType.DMA,
        pltpu.SemaphoreType.DMA,
    ],
)
def _sc_aggregate(fu, fi, s0, d0, s1, d1, s2, d2, za, irow,
                  sum0, cnt0, sum1, cnt1, sum2, cnt2,
                  src_v, dst_v, rows_v, lcnt, irow_v, tbl_sh, acc_sh, cnt_sh,
                  sem_a, sem_b, sem_c, sem_d, sem_e, sem_f, sem_g, sem_h):
    c = lax.axis_index("c")
    s = lax.axis_index("s")
    base = (c * NS + s) * TPW
    ones16 = jnp.full((16,), 1.0, _f32)
    sems = (sem_a, sem_b, sem_c, sem_d)
    ssems = (sem_e, sem_f, sem_g, sem_h)
    pltpu.sync_copy(irow, irow_v)
    for tbl, src_h, dst_h, sum_o, cnt_o in (
        (fu, s0, d0, sum0, cnt0),
        (fu, s1, d1, sum1, cnt1),
        (fi, s2, d2, sum2, cnt2),
    ):
        for h in (0, 1):
            pltpu.sync_copy(tbl.at[pl.ds(s * RPT, RPT), pl.ds(h * DH, DH)],
                            tbl_sh.at[pl.ds(s * RPT, RPT)])
            pltpu.sync_copy(za.at[pl.ds(0, ROWS_Z), pl.ds(0, DH)],
                            acc_sh.at[pl.ds(s * ROWS_Z, ROWS_Z)])
            if h == 0:
                pltpu.sync_copy(za.at[pl.ds(0, CR)], lcnt)

                @pl.when(s < CR // 8)
                def _():
                    pltpu.sync_copy(za.at[pl.ds(0, 8)],
                                    cnt_sh.at[pl.ds(s * 8, 8)])

            plsc.subcore_barrier()

            @pl.loop(0, TPW // IG)
            def _(g):
                off = base + g * IG
                pltpu.sync_copy(src_h.at[pl.ds(off, IG)], src_v)
                pltpu.sync_copy(dst_h.at[pl.ds(off, IG)], dst_v)
                desc = [None] * NBUF
                sdesc = [None] * NBUF
                for p in range(NBUF - 1):
                    desc[p] = pltpu.async_copy(tbl_sh.at[src_v.at[p]],
                                               rows_v.at[p], sems[p])
                for j in range(IG):
                    cur = j % NBUF
                    if j + NBUF - 1 < IG:
                        nxt = (j + NBUF - 1) % NBUF
                        if sdesc[nxt] is not None:
                            sdesc[nxt].wait()
                            sdesc[nxt] = None
                        desc[nxt] = pltpu.async_copy(
                            tbl_sh.at[src_v.at[j + NBUF - 1]], rows_v.at[nxt],
                            sems[nxt])
                    if h == 0:
                        for k in range(CHUNK // 16):
                            v = dst_v[j, pl.ds(k * 16, 16)]
                            row = lax.shift_right_logical(v, 7)
                            col = jnp.bitwise_and(v, 127)
                            plsc.addupdate_scatter(lcnt, [row, col], ones16)
                    desc[cur].wait()
                    sdesc[cur] = pltpu.async_copy(
                        rows_v.at[cur], acc_sh.at[dst_v.at[j]], ssems[cur],
                        add=True)
                for b in range(NBUF):
                    if sdesc[b] is not None:
                        sdesc[b].wait()

            if h == 0:
                pltpu.sync_copy(lcnt, cnt_sh.at[irow_v.at[0]], add=True)
            plsc.subcore_barrier()
            pltpu.sync_copy(acc_sh.at[pl.ds(s * ROWS_Z, ROWS_Z)],
                            sum_o.at[c, pl.ds(s * ROWS_Z, ROWS_Z),
                                     pl.ds(h * DH, DH)])
            if h == 0:

                @pl.when(s == 0)
                def _():
                    pltpu.sync_copy(cnt_sh, cnt_o.at[c])

            plsc.subcore_barrier()


BLK = 400


def _combine_body(sf, cf, srb, crb, sr, cr, wf, bf, wrb, brb, wr, br,
                  hu_ref, hi_ref):
    def mean_of(s_ref, c_ref):
        t = s_ref[0] + s_ref[1]
        cnt = c_ref[0] + c_ref[1]
        return t / jnp.maximum(cnt, 1.0), cnt

    dn = (((1,), (1,)), ((), ()))
    mf, cntf = mean_of(sf, cf)
    mrb, cntrb = mean_of(srb, crb)
    mr, cntr = mean_of(sr, cr)
    hu_ref[...] = (
        lax.dot_general(mf, wf[...], dn, preferred_element_type=_f32)
        + jnp.where(cntf > 0, bf[...], 0.0)
        + lax.dot_general(mrb, wrb[...], dn, preferred_element_type=_f32)
        + jnp.where(cntrb > 0, brb[...], 0.0)
    )
    hi_ref[...] = (
        lax.dot_general(mr, wr[...], dn, preferred_element_type=_f32)
        + jnp.where(cntr > 0, br[...], 0.0)
    )


def _combine(sf, cf, srb, crb, sr, cr, wf, bf, wrb, brb, wr, br):
    sum_spec = pl.BlockSpec((NC, BLK, D), lambda i: (0, i, 0))
    cnt_spec = pl.BlockSpec((NC, BLK, 1), lambda i: (0, i, 0))
    w_spec = pl.BlockSpec((D, D), lambda i: (0, 0))
    b_spec = pl.BlockSpec((1, D), lambda i: (0, 0))
    out_spec = pl.BlockSpec((BLK, D), lambda i: (i, 0))
    return pl.pallas_call(
        _combine_body,
        grid=(N // BLK,),
        in_specs=[sum_spec, cnt_spec, sum_spec, cnt_spec, sum_spec, cnt_spec,
                  w_spec, b_spec, w_spec, b_spec, w_spec, b_spec],
        out_specs=(out_spec, out_spec),
        out_shape=(jax.ShapeDtypeStruct((N, D), _f32),
                   jax.ShapeDtypeStruct((N, D), _f32)),
    )(sf, cf, srb, crb, sr, cr, wf, bf, wrb, brb, wr, br)


def kernel(feat_user, feat_item, W_follows, b_follows, W_rates, b_rates,
           W_rated_by, b_rated_by, src_follows, dst_follows, src_rates,
           dst_rates, src_rated_by, dst_rated_by):
    pad = E_PAD - E

    def prep(src, dst):
        s = jnp.concatenate([src, jnp.zeros((pad,), jnp.int32)])
        d = jnp.concatenate([dst, jnp.full((pad,), N, jnp.int32)])
        return s.reshape(NCH, CHUNK), d.reshape(NCH, CHUNK)

    s0, d0 = prep(src_follows, dst_follows)
    s1, d1 = prep(src_rates, dst_rates)
    s2, d2 = prep(src_rated_by, dst_rated_by)
    za = jnp.zeros((ROWS_Z, D), _f32)
    irow = jnp.arange(CR, dtype=jnp.int32).reshape(1, CR)

    sumf, cntf, sumr, cntr, sumrb, cntrb = _sc_aggregate(
        feat_user, feat_item,
        s0, d0, s1, d1, s2, d2, za, irow)

    def cview(cnt):
        return cnt.reshape(NC, CR * 128)[:, :N, None]

    h_user, h_item = _combine(
        sumf, cview(cntf), sumrb, cview(cntrb), sumr, cview(cntr),
        W_follows, b_follows.reshape(1, D),
        W_rated_by, b_rated_by.reshape(1, D),
        W_rates, b_rates.reshape(1, D))
    return (h_user, h_item)
